# baseline (speedup 1.0000x reference)
import functools

import jax
import jax.numpy as jnp
from jax import lax
from jax.experimental import pallas as pl
from jax.experimental.pallas import tpu as pltpu
from jax.experimental.pallas import tpu_sc as plsc

N = 10000
E = 320000
HID = 64
W = 128
XW = 16
NC, NS = 2, 16
NW = NC * NS
EPW = E // NW

EA = 192000
EB = E - EA
EPWA = EA // NW
EPWB = EB // NW

CHG = 80
GRPG = 5

CHS = 128
GRPS = 2
NACC = N + 8


def _silu(v):
    return v * jax.nn.sigmoid(v)


def _drain(hbm_ref, vmem_ref, sem):
    pltpu.make_async_copy(hbm_ref, vmem_ref, sem).wait()


@functools.cache
def _get_sc_gather(epw):
    nchg = epw // CHG
    ngrpg = nchg // GRPG
    ne = epw * NW

    def body(row_hbm, col_hbm, ta_hbm, tb_hbm, ga_hbm, gb_hbm,
             idxr, idxc, *rest):
        buf_a = rest[0:GRPG]
        buf_b = rest[GRPG:2 * GRPG]
        semg = rest[2 * GRPG:3 * GRPG]
        semw = rest[3 * GRPG:4 * GRPG]
        wid = lax.axis_index("s") * NC + lax.axis_index("c")
        ebase = wid * epw
        pltpu.sync_copy(row_hbm.at[pl.ds(ebase, epw)], idxr)
        pltpu.sync_copy(col_hbm.at[pl.ds(ebase, epw)], idxc)

        @pl.loop(0, ngrpg)
        def _grp(j):
            @pl.when(j > 0)
            def _():
                for s in range(GRPG):
                    _drain(ta_hbm.at[pl.ds(0, CHG)], buf_a[s], semw[s])
                    _drain(tb_hbm.at[pl.ds(0, CHG)], buf_b[s], semw[s])
            descs = []
            for s in range(GRPG):
                off = (j * GRPG + s) * CHG
                descs.append(pltpu.async_copy(
                    ta_hbm.at[idxr.at[pl.ds(off, CHG)]], buf_a[s], semg[s]))
                descs.append(pltpu.async_copy(
                    tb_hbm.at[idxc.at[pl.ds(off, CHG)]], buf_b[s], semg[s]))
            for s in range(GRPG):
                base = ebase + (j * GRPG + s) * CHG
                descs[2 * s].wait()
                descs[2 * s + 1].wait()
                pltpu.async_copy(buf_a[s], ga_hbm.at[pl.ds(base, CHG)], semw[s])
                pltpu.async_copy(buf_b[s], gb_hbm.at[pl.ds(base, CHG)], semw[s])

        for s in range(GRPG):
            _drain(ta_hbm.at[pl.ds(0, CHG)], buf_a[s], semw[s])
            _drain(tb_hbm.at[pl.ds(0, CHG)], buf_b[s], semw[s])

    mesh = plsc.VectorSubcoreMesh(
        core_axis_name="c", subcore_axis_name="s",
        num_cores=NC, num_subcores=NS)
    return pl.kernel(
        body,
        out_type=(
            jax.ShapeDtypeStruct((ne, W), jnp.float32),
            jax.ShapeDtypeStruct((ne, W), jnp.float32),
        ),
        mesh=mesh,
        scratch_types=(
            [pltpu.VMEM((epw,), jnp.int32)] * 2
            + [pltpu.VMEM((CHG, W), jnp.float32)] * (2 * GRPG)
            + [pltpu.SemaphoreType.DMA] * (2 * GRPG)
        ),
    )


def _sc_gather(epw, *args):
    return _get_sc_gather(epw)(*args)


def _seg_meta(epw):
    nfull = epw // CHS
    tail = epw - nfull * CHS
    spw = nfull + (1 if tail else 0)
    ngrps = nfull // GRPS
    rem = nfull - ngrps * GRPS
    return nfull, tail, spw, ngrps, rem


@functools.cache
def _get_sc_scatter():
    metaA = _seg_meta(EPWA)
    metaB = _seg_meta(EPWB)

    def body(rowpA_hbm, rowpB_hbm, oA_hbm, oB_hbm, zacc_hbm, p_hbm,
             acc, idxA, idxB, *rest):
        buf = rest[0:GRPS]
        seml = rest[GRPS:2 * GRPS]
        sema = rest[2 * GRPS:3 * GRPS]
        cid = lax.axis_index("c")
        sid = lax.axis_index("s")
        wid = sid * NC + cid

        @pl.when(sid == 0)
        def _():
            pltpu.sync_copy(zacc_hbm, acc)
        pltpu.sync_copy(rowpA_hbm.at[wid], idxA)
        pltpu.sync_copy(rowpB_hbm.at[wid], idxB)
        plsc.subcore_barrier()

        for o_hbm, idxr, epw, meta in (
                (oA_hbm, idxA, EPWA, metaA), (oB_hbm, idxB, EPWB, metaB)):
            nfull, tail, spw, ngrps, rem = meta
            ebase = wid * epw

            @pl.loop(0, ngrps)
            def _grp(j, o_hbm=o_hbm, idxr=idxr, ebase=ebase):
                @pl.when(j > 0)
                def _():
                    for s in range(GRPS):
                        _drain(o_hbm.at[pl.ds(0, CHS)], buf[s], sema[s])
                descs = []
                for s in range(GRPS):
                    base = ebase + (j * GRPS + s) * CHS
                    descs.append(pltpu.async_copy(
                        o_hbm.at[pl.ds(base, CHS)], buf[s], seml[s]))
                for s in range(GRPS):
                    ci = j * GRPS + s
                    descs[s].wait()
                    pltpu.async_copy(buf[s], acc.at[idxr.at[ci]], sema[s],
                                     add=True)

            for s in range(GRPS):
                _drain(o_hbm.at[pl.ds(0, CHS)], buf[s], sema[s])
            for k in range(rem):
                ci = ngrps * GRPS + k
                pltpu.sync_copy(o_hbm.at[pl.ds(ebase + ci * CHS, CHS)], buf[0])
                pltpu.sync_copy(buf[0], acc.at[idxr.at[ci]], add=True)
            if tail:
                pltpu.sync_copy(o_hbm.at[pl.ds(ebase + nfull * CHS, tail)],
                                buf[0].at[pl.ds(0, tail)])
                pltpu.sync_copy(buf[0], acc.at[idxr.at[nfull]], add=True)
        plsc.subcore_barrier()

        @pl.when(sid < 2)
        def _():
            r0 = sid * 632
            pltpu.sync_copy(acc.at[pl.ds(r0, 632)], p_hbm.at[cid, pl.ds(r0, 632)])

        @pl.when(sid >= 2)
        def _():
            r0 = 1264 + (sid - 2) * 624
            pltpu.sync_copy(acc.at[pl.ds(r0, 624)], p_hbm.at[cid, pl.ds(r0, 624)])

    mesh = plsc.VectorSubcoreMesh(
        core_axis_name="c", subcore_axis_name="s",
        num_cores=NC, num_subcores=NS)
    return pl.kernel(
        body,
        out_type=jax.ShapeDtypeStruct((NC, N, W), jnp.float32),
        mesh=mesh,
        scratch_types=(
            [pltpu.VMEM_SHARED((NACC, W), jnp.float32),
             pltpu.VMEM((metaA[2], CHS), jnp.int32),
             pltpu.VMEM((metaB[2], CHS), jnp.int32)]
            + [pltpu.VMEM((CHS, W), jnp.float32)] * GRPS
            + [pltpu.SemaphoreType.DMA] * (2 * GRPS)
        ),
    )


def _sc_scatter(*args):
    return _get_sc_scatter()(*args)


_NB = 2000
_NGRID = N // _NB
_EB = 6400
_EGRID = E // _EB


def _full(shape):
    return pl.BlockSpec(shape, lambda i: tuple(0 for _ in shape))


def _dot(a, b):
    return jnp.dot(a, b, preferred_element_type=jnp.float32)


def _embed_body(h_ref, w_ref, b_ref, o_ref):
    o_ref[...] = _dot(h_ref[...], w_ref[...]) + b_ref[...]


def _mm_call(in_nf, out_nf):
    return pl.pallas_call(
        _embed_body,
        grid=(_NGRID,),
        in_specs=[
            pl.BlockSpec((_NB, in_nf), lambda i: (i, 0)),
            _full((in_nf, out_nf)),
            _full((1, out_nf)),
        ],
        out_specs=pl.BlockSpec((_NB, out_nf), lambda i: (i, 0)),
        out_shape=jax.ShapeDtypeStruct((N, out_nf), jnp.float32),
    )


_embed_in = _mm_call(128, HID)
_embed_out = _mm_call(HID, 128)


def _prep_body(h_ref, x_ref, w0a_ref, w0b_ref, b0_ref, ta_ref, tb_ref):
    hv = h_ref[...]
    xv = x_ref[...]
    z = jnp.zeros((_NB, W - HID - XW), jnp.float32)
    ta_ref[...] = jnp.concatenate(
        [_dot(hv, w0a_ref[...]) + b0_ref[...], xv, z], axis=1)
    tb_ref[...] = jnp.concatenate([_dot(hv, w0b_ref[...]), -xv, z], axis=1)


_prep = pl.pallas_call(
    _prep_body,
    grid=(_NGRID,),
    in_specs=[
        pl.BlockSpec((_NB, HID), lambda i: (i, 0)),
        pl.BlockSpec((_NB, XW), lambda i: (i, 0)),
        _full((HID, HID)),
        _full((HID, HID)),
        _full((1, HID)),
    ],
    out_specs=[
        pl.BlockSpec((_NB, W), lambda i: (i, 0)),
        pl.BlockSpec((_NB, W), lambda i: (i, 0)),
    ],
    out_shape=[
        jax.ShapeDtypeStruct((N, W), jnp.float32),
        jax.ShapeDtypeStruct((N, W), jnp.float32),
    ],
)


def _edge_body(ga_ref, gb_ref, ea_ref,
               w0r_ref, w0e_ref, w1_ref, b1_ref, wc0_ref, bc0_ref, wc1_ref,
               o_ref):
    sv = ga_ref[...] + gb_ref[...]
    lane = lax.broadcasted_iota(jnp.int32, (_EB, W), 1)
    mx = ((lane >= HID) & (lane < HID + XW)).astype(jnp.float32)
    dm = sv * mx
    radial = jnp.sum(dm * dm, axis=1, keepdims=True)
    z1 = sv + radial * w0r_ref[...] + _dot(ea_ref[...], w0e_ref[...])
    e1 = _silu(z1)
    ef = _silu(_dot(e1, w1_ref[...]) + b1_ref[...])
    t = _silu(_dot(ef, wc0_ref[...]) + bc0_ref[...])
    cm = jnp.sum(t * wc1_ref[...], axis=1, keepdims=True)
    o_ref[...] = ef + dm * (cm / (jnp.sqrt(radial + 1e-8) + 1.0))


@functools.cache
def _get_edge(ne):
    return pl.pallas_call(
        _edge_body,
        grid=(ne // _EB,),
        in_specs=[
            pl.BlockSpec((_EB, W), lambda i: (i, 0)),
            pl.BlockSpec((_EB, W), lambda i: (i, 0)),
            pl.BlockSpec((_EB, 4), lambda i: (i, 0)),
            _full((1, W)),
            _full((4, W)),
            _full((W, W)),
            _full((1, W)),
            _full((W, HID)),
            _full((1, HID)),
            _full((1, HID)),
        ],
        out_specs=pl.BlockSpec((_EB, W), lambda i: (i, 0)),
        out_shape=jax.ShapeDtypeStruct((ne, W), jnp.float32),
    )


def _node_body(h_ref, p_ref, x_ref,
               wn0a_ref, wn0b_ref, bn0_ref, wn1_ref, bn1_ref,
               ho_ref, xo_ref):
    hv = h_ref[...]
    sp = p_ref[0] + p_ref[1]
    nm = _silu(_dot(hv, wn0a_ref[...]) + _dot(sp, wn0b_ref[...]) + bn0_ref[...])
    nm = _dot(nm, wn1_ref[...]) + bn1_ref[...]
    ho_ref[...] = hv + nm
    xo_ref[...] = x_ref[...] + sp[:, HID:HID + XW]


_node = pl.pallas_call(
    _node_body,
    grid=(_NGRID,),
    in_specs=[
        pl.BlockSpec((_NB, HID), lambda i: (i, 0)),
        pl.BlockSpec((NC, _NB, W), lambda i: (0, i, 0)),
        pl.BlockSpec((_NB, XW), lambda i: (i, 0)),
        _full((HID, HID)),
        _full((W, HID)),
        _full((1, HID)),
        _full((HID, HID)),
        _full((1, HID)),
    ],
    out_specs=[
        pl.BlockSpec((_NB, HID), lambda i: (i, 0)),
        pl.BlockSpec((_NB, XW), lambda i: (i, 0)),
    ],
    out_shape=[
        jax.ShapeDtypeStruct((N, HID), jnp.float32),
        jax.ShapeDtypeStruct((N, XW), jnp.float32),
    ],
)


def _pad_rowp(rowx, epw):
    spw = -(-epw // CHS)
    return jnp.pad(rowx.reshape(NW, epw), ((0, 0), (0, spw * CHS - epw)),
                   constant_values=N).reshape(NW, spw, CHS)


def kernel(h, x, edges, edge_attr, params):
    row = edges[0]
    col = edges[1]
    rowA, rowB = row[:EA], row[EA:]
    colA, colB = col[:EA], col[EA:]
    eaA, eaB = edge_attr[:EA], edge_attr[EA:]
    rowpA = _pad_rowp(rowA, EPWA)
    rowpB = _pad_rowp(rowB, EPWB)
    xp = jnp.pad(x, ((0, 0), (0, XW - 3)))
    zacc = jnp.zeros((NACC, W), jnp.float32)

    def b2d(b):
        return b.reshape(1, -1)

    def padw(m):
        return jnp.pad(m, ((0, 0), (0, W - m.shape[1])))

    hcur = _embed_in(h, params["emb_in"]["w"], b2d(params["emb_in"]["b"]))
    xcur = xp
    for lp in params["layers"]:
        w0 = lp["edge_mlp0"]["w"]
        w0a, w0b = w0[0:HID], w0[HID:2 * HID]
        w0r = padw(w0[2 * HID:2 * HID + 1])
        w0e = padw(w0[2 * HID + 1:])
        w1e = jnp.pad(lp["edge_mlp1"]["w"],
                      ((0, W - HID), (0, W - HID)))
        b1e = padw(b2d(lp["edge_mlp1"]["b"]))
        wc0e = jnp.pad(lp["coord_mlp0"]["w"], ((0, W - HID), (0, 0)))
        ta, tb = _prep(hcur, xcur, w0a, w0b, b2d(lp["edge_mlp0"]["b"]))
        mlp_args = (w0r, w0e, w1e, b1e,
                    wc0e, b2d(lp["coord_mlp0"]["b"]),
                    lp["coord_mlp1"]["w"].reshape(1, HID))
        gaA, gbA = _sc_gather(EPWA, rowA, colA, ta, tb)
        gaB, gbB = _sc_gather(EPWB, rowB, colB, ta, tb)
        oA = _get_edge(EA)(gaA, gbA, eaA, *mlp_args)
        oB = _get_edge(EB)(gaB, gbB, eaB, *mlp_args)
        p = _sc_scatter(rowpA, rowpB, oA, oB, zacc)
        nw0 = lp["node_mlp0"]["w"]
        wn0be = jnp.pad(nw0[HID:], ((0, W - HID), (0, 0)))
        hcur, xcur = _node(hcur, p, xcur,
                           nw0[:HID], wn0be, b2d(lp["node_mlp0"]["b"]),
                           lp["node_mlp1"]["w"], b2d(lp["node_mlp1"]["b"]))
    hout = _embed_out(hcur, params["emb_out"]["w"], b2d(params["emb_out"]["b"]))
    return (hout, xcur[:, :3])


# speedup vs baseline: 1.0563x; 1.0563x over previous
import functools

import jax
import jax.numpy as jnp
from jax import lax
from jax.experimental import pallas as pl
from jax.experimental.pallas import tpu as pltpu
from jax.experimental.pallas import tpu_sc as plsc

N = 10000
E = 320000
HID = 64
W = 128
XW = 16
NC, NS = 2, 16
NW = NC * NS
EPW = E // NW

EA = 192000
EB = E - EA
EPWA = EA // NW
EPWB = EB // NW

CHG = 80
GRPG = 5

CHS = 128
GRPS = 2
NACC = N + 8


def _silu(v):
    return v * jax.nn.sigmoid(v)


def _drain(hbm_ref, vmem_ref, sem):
    pltpu.make_async_copy(hbm_ref, vmem_ref, sem).wait()


@functools.cache
def _get_sc_gather(epw):
    nchg = epw // CHG
    ngrpg = nchg // GRPG
    ne = epw * NW

    def body(row_hbm, col_hbm, ta_hbm, tb_hbm, ga_hbm, gb_hbm,
             idxr, idxc, *rest):
        buf_a = rest[0:GRPG]
        buf_b = rest[GRPG:2 * GRPG]
        semg = rest[2 * GRPG:3 * GRPG]
        semw = rest[3 * GRPG:4 * GRPG]
        wid = lax.axis_index("s") * NC + lax.axis_index("c")
        ebase = wid * epw
        pltpu.sync_copy(row_hbm.at[pl.ds(ebase, epw)], idxr)
        pltpu.sync_copy(col_hbm.at[pl.ds(ebase, epw)], idxc)

        @pl.loop(0, ngrpg)
        def _grp(j):
            @pl.when(j > 0)
            def _():
                for s in range(GRPG):
                    _drain(ta_hbm.at[pl.ds(0, CHG)], buf_a[s], semw[s])
                    _drain(tb_hbm.at[pl.ds(0, CHG)], buf_b[s], semw[s])
            descs = []
            for s in range(GRPG):
                off = (j * GRPG + s) * CHG
                descs.append(pltpu.async_copy(
                    ta_hbm.at[idxr.at[pl.ds(off, CHG)]], buf_a[s], semg[s]))
                descs.append(pltpu.async_copy(
                    tb_hbm.at[idxc.at[pl.ds(off, CHG)]], buf_b[s], semg[s]))
            for s in range(GRPG):
                base = ebase + (j * GRPG + s) * CHG
                descs[2 * s].wait()
                descs[2 * s + 1].wait()
                pltpu.async_copy(buf_a[s], ga_hbm.at[pl.ds(base, CHG)], semw[s])
                pltpu.async_copy(buf_b[s], gb_hbm.at[pl.ds(base, CHG)], semw[s])

        for s in range(GRPG):
            _drain(ta_hbm.at[pl.ds(0, CHG)], buf_a[s], semw[s])
            _drain(tb_hbm.at[pl.ds(0, CHG)], buf_b[s], semw[s])

    mesh = plsc.VectorSubcoreMesh(
        core_axis_name="c", subcore_axis_name="s",
        num_cores=NC, num_subcores=NS)
    return pl.kernel(
        body,
        out_type=(
            jax.ShapeDtypeStruct((ne, W), jnp.float32),
            jax.ShapeDtypeStruct((ne, W), jnp.float32),
        ),
        mesh=mesh,
        scratch_types=(
            [pltpu.VMEM((epw,), jnp.int32)] * 2
            + [pltpu.VMEM((CHG, W), jnp.float32)] * (2 * GRPG)
            + [pltpu.SemaphoreType.DMA] * (2 * GRPG)
        ),
    )


def _sc_gather(epw, *args):
    return _get_sc_gather(epw)(*args)


@functools.cache
def _get_sc_scatter(epw):
    nfull = epw // CHS
    tail = epw - nfull * CHS
    spw = nfull + (1 if tail else 0)
    ngrps = nfull // GRPS
    rem = nfull - ngrps * GRPS

    def body(rowp_hbm, o_hbm, zacc_hbm, p_hbm, acc, idxr, *rest):
        buf = rest[0:GRPS]
        seml = rest[GRPS:2 * GRPS]
        sema = rest[2 * GRPS:3 * GRPS]
        cid = lax.axis_index("c")
        sid = lax.axis_index("s")
        wid = sid * NC + cid
        ebase = wid * epw

        @pl.when(sid == 0)
        def _():
            pltpu.sync_copy(zacc_hbm, acc)
        pltpu.sync_copy(rowp_hbm.at[wid], idxr)
        plsc.subcore_barrier()

        @pl.loop(0, ngrps)
        def _grp(j):
            @pl.when(j > 0)
            def _():
                for s in range(GRPS):
                    _drain(o_hbm.at[pl.ds(0, CHS)], buf[s], sema[s])
            descs = []
            for s in range(GRPS):
                base = ebase + (j * GRPS + s) * CHS
                descs.append(
                    pltpu.async_copy(o_hbm.at[pl.ds(base, CHS)], buf[s], seml[s]))
            for s in range(GRPS):
                ci = j * GRPS + s
                descs[s].wait()
                pltpu.async_copy(buf[s], acc.at[idxr.at[ci]], sema[s], add=True)

        for s in range(GRPS):
            _drain(o_hbm.at[pl.ds(0, CHS)], buf[s], sema[s])
        for k in range(rem):
            ci = ngrps * GRPS + k
            pltpu.sync_copy(o_hbm.at[pl.ds(ebase + ci * CHS, CHS)], buf[0])
            pltpu.sync_copy(buf[0], acc.at[idxr.at[ci]], add=True)
        if tail:
            pltpu.sync_copy(o_hbm.at[pl.ds(ebase + nfull * CHS, tail)],
                            buf[0].at[pl.ds(0, tail)])
            pltpu.sync_copy(buf[0], acc.at[idxr.at[nfull]], add=True)
        plsc.subcore_barrier()

        @pl.when(sid < 2)
        def _():
            r0 = sid * 632
            pltpu.sync_copy(acc.at[pl.ds(r0, 632)], p_hbm.at[cid, pl.ds(r0, 632)])

        @pl.when(sid >= 2)
        def _():
            r0 = 1264 + (sid - 2) * 624
            pltpu.sync_copy(acc.at[pl.ds(r0, 624)], p_hbm.at[cid, pl.ds(r0, 624)])

    mesh = plsc.VectorSubcoreMesh(
        core_axis_name="c", subcore_axis_name="s",
        num_cores=NC, num_subcores=NS)
    return pl.kernel(
        body,
        out_type=jax.ShapeDtypeStruct((NC, N, W), jnp.float32),
        mesh=mesh,
        scratch_types=(
            [pltpu.VMEM_SHARED((NACC, W), jnp.float32),
             pltpu.VMEM((spw, CHS), jnp.int32)]
            + [pltpu.VMEM((CHS, W), jnp.float32)] * GRPS
            + [pltpu.SemaphoreType.DMA] * (2 * GRPS)
        ),
    )


def _sc_scatter(epw, *args):
    return _get_sc_scatter(epw)(*args)


_NB = 2000
_NGRID = N // _NB
_EB = 6400
_EGRID = E // _EB


def _full(shape):
    return pl.BlockSpec(shape, lambda i: tuple(0 for _ in shape))


def _dot(a, b):
    return jnp.dot(a, b, preferred_element_type=jnp.float32)


def _embed_body(h_ref, w_ref, b_ref, o_ref):
    o_ref[...] = _dot(h_ref[...], w_ref[...]) + b_ref[...]


def _mm_call(in_nf, out_nf):
    return pl.pallas_call(
        _embed_body,
        grid=(_NGRID,),
        in_specs=[
            pl.BlockSpec((_NB, in_nf), lambda i: (i, 0)),
            _full((in_nf, out_nf)),
            _full((1, out_nf)),
        ],
        out_specs=pl.BlockSpec((_NB, out_nf), lambda i: (i, 0)),
        out_shape=jax.ShapeDtypeStruct((N, out_nf), jnp.float32),
    )


_embed_in = _mm_call(128, HID)
_embed_out = _mm_call(HID, 128)


def _prep_body(h_ref, x_ref, w0a_ref, w0b_ref, b0_ref, ta_ref, tb_ref):
    hv = h_ref[...]
    xv = x_ref[...]
    z = jnp.zeros((_NB, W - HID - XW), jnp.float32)
    ta_ref[...] = jnp.concatenate(
        [_dot(hv, w0a_ref[...]) + b0_ref[...], xv, z], axis=1)
    tb_ref[...] = jnp.concatenate([_dot(hv, w0b_ref[...]), -xv, z], axis=1)


_prep = pl.pallas_call(
    _prep_body,
    grid=(_NGRID,),
    in_specs=[
        pl.BlockSpec((_NB, HID), lambda i: (i, 0)),
        pl.BlockSpec((_NB, XW), lambda i: (i, 0)),
        _full((HID, HID)),
        _full((HID, HID)),
        _full((1, HID)),
    ],
    out_specs=[
        pl.BlockSpec((_NB, W), lambda i: (i, 0)),
        pl.BlockSpec((_NB, W), lambda i: (i, 0)),
    ],
    out_shape=[
        jax.ShapeDtypeStruct((N, W), jnp.float32),
        jax.ShapeDtypeStruct((N, W), jnp.float32),
    ],
)


def _edge_body(ga_ref, gb_ref, ea_ref,
               w0r_ref, w0e_ref, w1_ref, b1_ref, wc0_ref, bc0_ref, wc1_ref,
               o_ref):
    sv = ga_ref[...] + gb_ref[...]
    lane = lax.broadcasted_iota(jnp.int32, (_EB, W), 1)
    mx = ((lane >= HID) & (lane < HID + XW)).astype(jnp.float32)
    dm = sv * mx
    radial = jnp.sum(dm * dm, axis=1, keepdims=True)
    z1 = sv + radial * w0r_ref[...] + _dot(ea_ref[...], w0e_ref[...])
    e1 = _silu(z1)
    ef = _silu(_dot(e1, w1_ref[...]) + b1_ref[...])
    t = _silu(_dot(ef, wc0_ref[...]) + bc0_ref[...])
    cm = jnp.sum(t * wc1_ref[...], axis=1, keepdims=True)
    o_ref[...] = ef + dm * (cm / (jnp.sqrt(radial + 1e-8) + 1.0))


@functools.cache
def _get_edge(ne):
    return pl.pallas_call(
        _edge_body,
        grid=(ne // _EB,),
        in_specs=[
            pl.BlockSpec((_EB, W), lambda i: (i, 0)),
            pl.BlockSpec((_EB, W), lambda i: (i, 0)),
            pl.BlockSpec((_EB, 4), lambda i: (i, 0)),
            _full((1, W)),
            _full((4, W)),
            _full((W, W)),
            _full((1, W)),
            _full((W, HID)),
            _full((1, HID)),
            _full((1, HID)),
        ],
        out_specs=pl.BlockSpec((_EB, W), lambda i: (i, 0)),
        out_shape=jax.ShapeDtypeStruct((ne, W), jnp.float32),
    )


def _node_body(h_ref, p_ref, q_ref, x_ref,
               wn0a_ref, wn0b_ref, bn0_ref, wn1_ref, bn1_ref,
               ho_ref, xo_ref):
    hv = h_ref[...]
    sp = (p_ref[0] + p_ref[1]) + (q_ref[0] + q_ref[1])
    nm = _silu(_dot(hv, wn0a_ref[...]) + _dot(sp, wn0b_ref[...]) + bn0_ref[...])
    nm = _dot(nm, wn1_ref[...]) + bn1_ref[...]
    ho_ref[...] = hv + nm
    xo_ref[...] = x_ref[...] + sp[:, HID:HID + XW]


_node = pl.pallas_call(
    _node_body,
    grid=(_NGRID,),
    in_specs=[
        pl.BlockSpec((_NB, HID), lambda i: (i, 0)),
        pl.BlockSpec((NC, _NB, W), lambda i: (0, i, 0)),
        pl.BlockSpec((NC, _NB, W), lambda i: (0, i, 0)),
        pl.BlockSpec((_NB, XW), lambda i: (i, 0)),
        _full((HID, HID)),
        _full((W, HID)),
        _full((1, HID)),
        _full((HID, HID)),
        _full((1, HID)),
    ],
    out_specs=[
        pl.BlockSpec((_NB, HID), lambda i: (i, 0)),
        pl.BlockSpec((_NB, XW), lambda i: (i, 0)),
    ],
    out_shape=[
        jax.ShapeDtypeStruct((N, HID), jnp.float32),
        jax.ShapeDtypeStruct((N, XW), jnp.float32),
    ],
)


def _pad_rowp(rowx, epw):
    spw = -(-epw // CHS)
    return jnp.pad(rowx.reshape(NW, epw), ((0, 0), (0, spw * CHS - epw)),
                   constant_values=N).reshape(NW, spw, CHS)


def kernel(h, x, edges, edge_attr, params):
    row = edges[0]
    col = edges[1]
    rowA, rowB = row[:EA], row[EA:]
    colA, colB = col[:EA], col[EA:]
    eaA, eaB = edge_attr[:EA], edge_attr[EA:]
    rowpA = _pad_rowp(rowA, EPWA)
    rowpB = _pad_rowp(rowB, EPWB)
    xp = jnp.pad(x, ((0, 0), (0, XW - 3)))
    zacc = jnp.zeros((NACC, W), jnp.float32)

    def b2d(b):
        return b.reshape(1, -1)

    def padw(m):
        return jnp.pad(m, ((0, 0), (0, W - m.shape[1])))

    hcur = _embed_in(h, params["emb_in"]["w"], b2d(params["emb_in"]["b"]))
    xcur = xp
    for lp in params["layers"]:
        w0 = lp["edge_mlp0"]["w"]
        w0a, w0b = w0[0:HID], w0[HID:2 * HID]
        w0r = padw(w0[2 * HID:2 * HID + 1])
        w0e = padw(w0[2 * HID + 1:])
        w1e = jnp.pad(lp["edge_mlp1"]["w"],
                      ((0, W - HID), (0, W - HID)))
        b1e = padw(b2d(lp["edge_mlp1"]["b"]))
        wc0e = jnp.pad(lp["coord_mlp0"]["w"], ((0, W - HID), (0, 0)))
        ta, tb = _prep(hcur, xcur, w0a, w0b, b2d(lp["edge_mlp0"]["b"]))
        mlp_args = (w0r, w0e, w1e, b1e,
                    wc0e, b2d(lp["coord_mlp0"]["b"]),
                    lp["coord_mlp1"]["w"].reshape(1, HID))
        gaA, gbA = _sc_gather(EPWA, rowA, colA, ta, tb)
        gaB, gbB = _sc_gather(EPWB, rowB, colB, ta, tb)
        oA = _get_edge(EA)(gaA, gbA, eaA, *mlp_args)
        oB = _get_edge(EB)(gaB, gbB, eaB, *mlp_args)
        pA = _sc_scatter(EPWA, rowpA, oA, zacc)
        pB = _sc_scatter(EPWB, rowpB, oB, zacc)
        nw0 = lp["node_mlp0"]["w"]
        wn0be = jnp.pad(nw0[HID:], ((0, W - HID), (0, 0)))
        hcur, xcur = _node(hcur, pA, pB, xcur,
                           nw0[:HID], wn0be, b2d(lp["node_mlp0"]["b"]),
                           lp["node_mlp1"]["w"], b2d(lp["node_mlp1"]["b"]))
    hout = _embed_out(hcur, params["emb_out"]["w"], b2d(params["emb_out"]["b"]))
    return (hout, xcur[:, :3])
